# baseline (device time: 197854 ns/iter reference)
import jax
import jax.numpy as jnp
from jax import lax
from jax.experimental import pallas as pl
from jax.experimental.pallas import tpu as pltpu


def kernel(O, Wo):
    B, S, H, D = O.shape
    HD = H * D
    N = Wo.shape[1]
    s_half = S // 2
    n_half = N // 2

    O = O.reshape(B, S, HD).astype(jnp.bfloat16)
    Wo = Wo.astype(jnp.bfloat16)

    def body(o_hbm, w_hbm, out_hbm,
             w_buf, o_slots, send_buf, rx_buf, pown_buf,
             sx_sems, rx_sems, sy_sems, ry_sems,
             w_sem, load_sems, store_sem):
        my_x = lax.axis_index("x")
        my_y = lax.axis_index("y")
        x_peer = (1 - my_x, my_y)
        y_peer = (my_x, 1 - my_y)

        barrier = pltpu.get_barrier_semaphore()
        for nbr in (x_peer, y_peer):
            pl.semaphore_signal(
                barrier, inc=1, device_id=nbr,
                device_id_type=pl.DeviceIdType.MESH,
            )
        pl.semaphore_wait(barrier, 2)

        peer_s0 = (1 - my_x) * s_half
        my_s0 = my_x * s_half
        col0 = my_y * n_half

        wcp = pltpu.make_async_copy(
            w_hbm.at[:, pl.ds(col0, n_half)], w_buf, w_sem,
        )
        wcp.start()

        loads = [(b, peer_s0) for b in range(B)] + [(b, my_s0) for b in range(B)]

        def start_load(i):
            b, s0 = loads[i]
            cp = pltpu.make_async_copy(
                o_hbm.at[b, pl.ds(s0, s_half), :],
                o_slots.at[i % 2], load_sems.at[i % 2],
            )
            cp.start()
            return cp

        pending = {0: start_load(0)}
        wcp.wait()

        x_rdmas = []
        for b in range(B):
            pending[b + 1] = start_load(b + 1)
            pending.pop(b).wait()
            p = jnp.dot(o_slots[b % 2], w_buf[...],
                        preferred_element_type=jnp.float32)
            send_buf[b, :, :] = p.astype(jnp.bfloat16)
            rdma = pltpu.make_async_remote_copy(
                src_ref=send_buf.at[b],
                dst_ref=rx_buf.at[b],
                send_sem=sx_sems.at[b],
                recv_sem=rx_sems.at[b],
                device_id=x_peer,
                device_id_type=pl.DeviceIdType.MESH,
            )
            rdma.start()
            x_rdmas.append(rdma)

        for b in range(B):
            i = B + b
            if i + 1 < 2 * B:
                pending[i + 1] = start_load(i + 1)
            pending.pop(i).wait()
            p = jnp.dot(o_slots[i % 2], w_buf[...],
                        preferred_element_type=jnp.float32)
            pown_buf[b, :, :] = p.astype(jnp.bfloat16)

        y_rdmas = []
        for b in range(B):
            x_rdmas[b].wait()
            send_buf[b, :, :] = pown_buf[b, :, :] + rx_buf[b, :, :]
            st = pltpu.make_async_copy(
                send_buf.at[b], out_hbm.at[b, :, pl.ds(col0, n_half)],
                store_sem,
            )
            st.start()
            st.wait()
            rdma = pltpu.make_async_remote_copy(
                src_ref=send_buf.at[b],
                dst_ref=out_hbm.at[b, :, pl.ds(col0, n_half)],
                send_sem=sy_sems.at[b],
                recv_sem=ry_sems.at[b],
                device_id=y_peer,
                device_id_type=pl.DeviceIdType.MESH,
            )
            rdma.start()
            y_rdmas.append(rdma)

        for b in range(B):
            y_rdmas[b].wait()

    out = pl.pallas_call(
        body,
        out_shape=jax.ShapeDtypeStruct((B, s_half, N), jnp.bfloat16),
        in_specs=[
            pl.BlockSpec(memory_space=pl.ANY),
            pl.BlockSpec(memory_space=pl.ANY),
        ],
        out_specs=pl.BlockSpec(memory_space=pl.ANY),
        scratch_shapes=[
            pltpu.VMEM((HD, n_half), jnp.bfloat16),
            pltpu.VMEM((2, s_half, HD), jnp.bfloat16),
            pltpu.VMEM((B, s_half, n_half), jnp.bfloat16),
            pltpu.VMEM((B, s_half, n_half), jnp.bfloat16),
            pltpu.VMEM((B, s_half, n_half), jnp.bfloat16),
            pltpu.SemaphoreType.DMA((B,)),
            pltpu.SemaphoreType.DMA((B,)),
            pltpu.SemaphoreType.DMA((B,)),
            pltpu.SemaphoreType.DMA((B,)),
            pltpu.SemaphoreType.DMA,
            pltpu.SemaphoreType.DMA((2,)),
            pltpu.SemaphoreType.DMA,
        ],
        compiler_params=pltpu.CompilerParams(
            collective_id=0,
            vmem_limit_bytes=100 * 1024 * 1024,
        ),
    )(O, Wo)
    return out


# device time: 192742 ns/iter; 1.0265x vs baseline; 1.0265x over previous
import jax
import jax.numpy as jnp
from jax import lax
from jax.experimental import pallas as pl
from jax.experimental.pallas import tpu as pltpu


def kernel(O, Wo):
    B, S, H, D = O.shape
    HD = H * D
    N = Wo.shape[1]
    s_half = S // 2
    n_half = N // 2
    hd_half = HD // 2

    O = O.reshape(B, S, HD)

    def body(o_hbm, w_hbm, out_hbm,
             w_stage, w_buf, o_slots, send_buf, rx_buf, pown_buf,
             sx_sems, rx_sems, sy_sems, ry_sems,
             w_sem, load_sems, store_sem):
        my_x = lax.axis_index("x")
        my_y = lax.axis_index("y")
        x_peer = (1 - my_x, my_y)
        y_peer = (my_x, 1 - my_y)

        barrier = pltpu.get_barrier_semaphore()
        for nbr in (x_peer, y_peer):
            pl.semaphore_signal(
                barrier, inc=1, device_id=nbr,
                device_id_type=pl.DeviceIdType.MESH,
            )
        pl.semaphore_wait(barrier, 2)

        peer_s0 = (1 - my_x) * s_half
        my_s0 = my_x * s_half
        col0 = my_y * n_half

        loads = [(b, peer_s0) for b in range(B)] + [(b, my_s0) for b in range(B)]

        def start_load(i):
            b, s0 = loads[i]
            cp = pltpu.make_async_copy(
                o_hbm.at[b, pl.ds(s0, s_half), :],
                o_slots.at[i % 2], load_sems.at[i % 2],
            )
            cp.start()
            return cp

        pending = {0: start_load(0)}

        for c in range(2):
            wcp = pltpu.make_async_copy(
                w_hbm.at[pl.ds(c * hd_half, hd_half), pl.ds(col0, n_half)],
                w_stage, w_sem,
            )
            wcp.start()
            wcp.wait()
            w_buf[c * hd_half:(c + 1) * hd_half, :] = (
                w_stage[...].astype(jnp.bfloat16))

        x_rdmas = []
        for b in range(B):
            pending[b + 1] = start_load(b + 1)
            pending.pop(b).wait()
            p = jnp.dot(o_slots[b % 2].astype(jnp.bfloat16), w_buf[...],
                        preferred_element_type=jnp.float32)
            send_buf[b, :, :] = p.astype(jnp.bfloat16)
            rdma = pltpu.make_async_remote_copy(
                src_ref=send_buf.at[b],
                dst_ref=rx_buf.at[b],
                send_sem=sx_sems.at[b],
                recv_sem=rx_sems.at[b],
                device_id=x_peer,
                device_id_type=pl.DeviceIdType.MESH,
            )
            rdma.start()
            x_rdmas.append(rdma)

        for b in range(B):
            i = B + b
            if i + 1 < 2 * B:
                pending[i + 1] = start_load(i + 1)
            pending.pop(i).wait()
            p = jnp.dot(o_slots[i % 2].astype(jnp.bfloat16), w_buf[...],
                        preferred_element_type=jnp.float32)
            pown_buf[b, :, :] = p.astype(jnp.bfloat16)

        y_rdmas = []
        for b in range(B):
            x_rdmas[b].wait()
            send_buf[b, :, :] = pown_buf[b, :, :] + rx_buf[b, :, :]
            st = pltpu.make_async_copy(
                send_buf.at[b], out_hbm.at[b, :, pl.ds(col0, n_half)],
                store_sem,
            )
            st.start()
            st.wait()
            rdma = pltpu.make_async_remote_copy(
                src_ref=send_buf.at[b],
                dst_ref=out_hbm.at[b, :, pl.ds(col0, n_half)],
                send_sem=sy_sems.at[b],
                recv_sem=ry_sems.at[b],
                device_id=y_peer,
                device_id_type=pl.DeviceIdType.MESH,
            )
            rdma.start()
            y_rdmas.append(rdma)

        for b in range(B):
            y_rdmas[b].wait()

    out = pl.pallas_call(
        body,
        out_shape=jax.ShapeDtypeStruct((B, s_half, N), jnp.bfloat16),
        in_specs=[
            pl.BlockSpec(memory_space=pl.ANY),
            pl.BlockSpec(memory_space=pl.ANY),
        ],
        out_specs=pl.BlockSpec(memory_space=pl.ANY),
        scratch_shapes=[
            pltpu.VMEM((hd_half, n_half), jnp.float32),
            pltpu.VMEM((HD, n_half), jnp.bfloat16),
            pltpu.VMEM((2, s_half, HD), jnp.float32),
            pltpu.VMEM((B, s_half, n_half), jnp.bfloat16),
            pltpu.VMEM((B, s_half, n_half), jnp.bfloat16),
            pltpu.VMEM((B, s_half, n_half), jnp.bfloat16),
            pltpu.SemaphoreType.DMA((B,)),
            pltpu.SemaphoreType.DMA((B,)),
            pltpu.SemaphoreType.DMA((B,)),
            pltpu.SemaphoreType.DMA((B,)),
            pltpu.SemaphoreType.DMA,
            pltpu.SemaphoreType.DMA((2,)),
            pltpu.SemaphoreType.DMA,
        ],
        compiler_params=pltpu.CompilerParams(
            collective_id=0,
            vmem_limit_bytes=100 * 1024 * 1024,
        ),
    )(O, Wo)
    return out


# device time: 187846 ns/iter; 1.0533x vs baseline; 1.0261x over previous
import jax
import jax.numpy as jnp
from jax import lax
from jax.experimental import pallas as pl
from jax.experimental.pallas import tpu as pltpu


def kernel(O, Wo):
    B, S, H, D = O.shape
    HD = H * D
    N = Wo.shape[1]
    s_half = S // 2
    n_half = N // 2
    hd_half = HD // 2

    O = O.reshape(B, S, HD)

    def body(o_hbm, w_hbm, out_hbm,
             w_stage, w_buf, o_slots, send_buf, rx_buf,
             sx_sems, rx_sems, sy_sems, ry_sems,
             w_sem, load_sems, store_sem):
        my_x = lax.axis_index("x")
        my_y = lax.axis_index("y")
        x_peer = (1 - my_x, my_y)
        y_peer = (my_x, 1 - my_y)

        barrier = pltpu.get_barrier_semaphore()
        for nbr in (x_peer, y_peer):
            pl.semaphore_signal(
                barrier, inc=1, device_id=nbr,
                device_id_type=pl.DeviceIdType.MESH,
            )
        pl.semaphore_wait(barrier, 2)

        peer_s0 = (1 - my_x) * s_half
        my_s0 = my_x * s_half
        col0 = my_y * n_half

        loads = [(b, peer_s0) for b in range(B)] + [(b, my_s0) for b in range(B)]

        def start_load(i):
            b, s0 = loads[i]
            cp = pltpu.make_async_copy(
                o_hbm.at[b, pl.ds(s0, s_half), :],
                o_slots.at[i % 2], load_sems.at[i % 2],
            )
            cp.start()
            return cp

        pending = {0: start_load(0)}

        for c in range(2):
            wcp = pltpu.make_async_copy(
                w_hbm.at[pl.ds(c * hd_half, hd_half), pl.ds(col0, n_half)],
                w_stage, w_sem,
            )
            wcp.start()
            wcp.wait()
            w_buf[c * hd_half:(c + 1) * hd_half, :] = (
                w_stage[...].astype(jnp.bfloat16))

        x_rdmas = []
        for b in range(B):
            pending[b + 1] = start_load(b + 1)
            pending.pop(b).wait()
            p = jnp.dot(o_slots[b % 2].astype(jnp.bfloat16), w_buf[...],
                        preferred_element_type=jnp.float32)
            send_buf[b, :, :] = p.astype(jnp.bfloat16)
            rdma = pltpu.make_async_remote_copy(
                src_ref=send_buf.at[b],
                dst_ref=rx_buf.at[b],
                send_sem=sx_sems.at[b],
                recv_sem=rx_sems.at[b],
                device_id=x_peer,
                device_id_type=pl.DeviceIdType.MESH,
            )
            rdma.start()
            x_rdmas.append(rdma)

        y_rdmas = []
        for b in range(B):
            i = B + b
            if i + 1 < 2 * B:
                pending[i + 1] = start_load(i + 1)
            pending.pop(i).wait()
            p = jnp.dot(o_slots[i % 2].astype(jnp.bfloat16), w_buf[...],
                        preferred_element_type=jnp.float32)
            x_rdmas[b].wait()
            send_buf[b, :, :] = (p + rx_buf[b, :, :].astype(jnp.float32)
                                 ).astype(jnp.bfloat16)
            st = pltpu.make_async_copy(
                send_buf.at[b], out_hbm.at[b, :, pl.ds(col0, n_half)],
                store_sem,
            )
            st.start()
            st.wait()
            rdma = pltpu.make_async_remote_copy(
                src_ref=send_buf.at[b],
                dst_ref=out_hbm.at[b, :, pl.ds(col0, n_half)],
                send_sem=sy_sems.at[b],
                recv_sem=ry_sems.at[b],
                device_id=y_peer,
                device_id_type=pl.DeviceIdType.MESH,
            )
            rdma.start()
            y_rdmas.append(rdma)

        for b in range(B):
            y_rdmas[b].wait()

    out = pl.pallas_call(
        body,
        out_shape=jax.ShapeDtypeStruct((B, s_half, N), jnp.bfloat16),
        in_specs=[
            pl.BlockSpec(memory_space=pl.ANY),
            pl.BlockSpec(memory_space=pl.ANY),
        ],
        out_specs=pl.BlockSpec(memory_space=pl.ANY),
        scratch_shapes=[
            pltpu.VMEM((hd_half, n_half), jnp.float32),
            pltpu.VMEM((HD, n_half), jnp.bfloat16),
            pltpu.VMEM((2, s_half, HD), jnp.float32),
            pltpu.VMEM((B, s_half, n_half), jnp.bfloat16),
            pltpu.VMEM((B, s_half, n_half), jnp.bfloat16),
            pltpu.SemaphoreType.DMA((B,)),
            pltpu.SemaphoreType.DMA((B,)),
            pltpu.SemaphoreType.DMA((B,)),
            pltpu.SemaphoreType.DMA((B,)),
            pltpu.SemaphoreType.DMA,
            pltpu.SemaphoreType.DMA((2,)),
            pltpu.SemaphoreType.DMA,
        ],
        compiler_params=pltpu.CompilerParams(
            collective_id=0,
            vmem_limit_bytes=100 * 1024 * 1024,
        ),
    )(O, Wo)
    return out


# device time: 179080 ns/iter; 1.1048x vs baseline; 1.0490x over previous
import jax
import jax.numpy as jnp
from jax import lax
from jax.experimental import pallas as pl
from jax.experimental.pallas import tpu as pltpu


def kernel(O, Wo):
    B, S, H, D = O.shape
    HD = H * D
    N = Wo.shape[1]
    s_half = S // 2
    n_half = N // 2
    hd_half = HD // 2
    s_sub = s_half // 2

    O = O.reshape(B, S, HD)

    def body(o_hbm, w_hbm, out_hbm,
             w_stage, w_buf, o_slots, send_buf, rx_buf,
             sx_sems, rx_sems, sy_sems, ry_sems,
             w_sems, load_sems, store_sems):
        my_x = lax.axis_index("x")
        my_y = lax.axis_index("y")
        x_peer = (1 - my_x, my_y)
        y_peer = (my_x, 1 - my_y)

        peer_s0 = (1 - my_x) * s_half
        my_s0 = my_x * s_half
        col0 = my_y * n_half

        loads = [(b, peer_s0) for b in range(B)] + [(b, my_s0) for b in range(B)]

        def start_load(i):
            b, s0 = loads[i]
            cp = pltpu.make_async_copy(
                o_hbm.at[b, pl.ds(s0, s_half), :],
                o_slots.at[i % 2], load_sems.at[i % 2],
            )
            cp.start()
            return cp

        wcps = []
        for c in range(2):
            wcp = pltpu.make_async_copy(
                w_hbm.at[pl.ds(c * hd_half, hd_half), pl.ds(col0, n_half)],
                w_stage.at[c], w_sems.at[c],
            )
            wcp.start()
            wcps.append(wcp)
        pending = {0: start_load(0)}

        barrier = pltpu.get_barrier_semaphore()
        for nbr in (x_peer, y_peer):
            pl.semaphore_signal(
                barrier, inc=1, device_id=nbr,
                device_id_type=pl.DeviceIdType.MESH,
            )
        pl.semaphore_wait(barrier, 2)

        for c in range(2):
            wcps[c].wait()
            w_buf[c * hd_half:(c + 1) * hd_half, :] = (
                w_stage[c].astype(jnp.bfloat16))

        x_rdmas = []
        for b in range(B):
            pending[b + 1] = start_load(b + 1)
            pending.pop(b).wait()
            p = jnp.dot(o_slots[b % 2].astype(jnp.bfloat16), w_buf[...],
                        preferred_element_type=jnp.float32)
            send_buf[b, :, :] = p.astype(jnp.bfloat16)
            rdma = pltpu.make_async_remote_copy(
                src_ref=send_buf.at[b],
                dst_ref=rx_buf.at[b],
                send_sem=sx_sems.at[b],
                recv_sem=rx_sems.at[b],
                device_id=x_peer,
                device_id_type=pl.DeviceIdType.MESH,
            )
            rdma.start()
            x_rdmas.append(rdma)

        y_rdmas = []
        stores = []
        for b in range(B):
            i = B + b
            if i + 1 < 2 * B:
                pending[i + 1] = start_load(i + 1)
            pending.pop(i).wait()
            p = jnp.dot(o_slots[i % 2].astype(jnp.bfloat16), w_buf[...],
                        preferred_element_type=jnp.float32)
            x_rdmas[b].wait()
            for k in range(2):
                rows = pl.ds(k * s_sub, s_sub)
                send_buf[b, rows, :] = (
                    p[k * s_sub:(k + 1) * s_sub, :]
                    + rx_buf[b, rows, :].astype(jnp.float32)
                ).astype(jnp.bfloat16)
                rdma = pltpu.make_async_remote_copy(
                    src_ref=send_buf.at[b, rows],
                    dst_ref=out_hbm.at[b, rows, pl.ds(col0, n_half)],
                    send_sem=sy_sems.at[2 * b + k],
                    recv_sem=ry_sems.at[2 * b + k],
                    device_id=y_peer,
                    device_id_type=pl.DeviceIdType.MESH,
                )
                rdma.start()
                y_rdmas.append(rdma)
            st = pltpu.make_async_copy(
                send_buf.at[b], out_hbm.at[b, :, pl.ds(col0, n_half)],
                store_sems.at[b],
            )
            st.start()
            stores.append(st)

        for rdma in y_rdmas:
            rdma.wait()
        for st in stores:
            st.wait()

    out = pl.pallas_call(
        body,
        out_shape=jax.ShapeDtypeStruct((B, s_half, N), jnp.bfloat16),
        in_specs=[
            pl.BlockSpec(memory_space=pl.ANY),
            pl.BlockSpec(memory_space=pl.ANY),
        ],
        out_specs=pl.BlockSpec(memory_space=pl.ANY),
        scratch_shapes=[
            pltpu.VMEM((2, hd_half, n_half), jnp.float32),
            pltpu.VMEM((HD, n_half), jnp.bfloat16),
            pltpu.VMEM((2, s_half, HD), jnp.float32),
            pltpu.VMEM((B, s_half, n_half), jnp.bfloat16),
            pltpu.VMEM((B, s_half, n_half), jnp.bfloat16),
            pltpu.SemaphoreType.DMA((B,)),
            pltpu.SemaphoreType.DMA((B,)),
            pltpu.SemaphoreType.DMA((2 * B,)),
            pltpu.SemaphoreType.DMA((2 * B,)),
            pltpu.SemaphoreType.DMA((2,)),
            pltpu.SemaphoreType.DMA((2,)),
            pltpu.SemaphoreType.DMA((B,)),
        ],
        compiler_params=pltpu.CompilerParams(
            collective_id=0,
            vmem_limit_bytes=100 * 1024 * 1024,
        ),
    )(O, Wo)
    return out
